# Initial kernel scaffold; baseline (speedup 1.0000x reference)
import jax
import jax.numpy as jnp
from jax.experimental import pallas as pl


def kernel(pos):
    raise NotImplementedError("write your pallas kernel here")


# speedup vs baseline: 10.8909x; 10.8909x over previous
import functools

import jax
import jax.numpy as jnp
from jax import lax
from jax.experimental import pallas as pl

N = 4096
K = 9
KPAD = 16
BIG = 3.0e38


def _knn_body(pos_ref, posT_ref, vals_ref, idx_ref, *, rows):
    xb = pos_ref[:, 0:1]
    yb = pos_ref[:, 1:2]
    zb = pos_ref[:, 2:3]
    x = posT_ref[0:1, :]
    y = posT_ref[1:2, :]
    z = posT_ref[2:3, :]
    sqb = xb * xb + yb * yb + zb * zb
    sq = x * x + y * y + z * z
    def b16(v):
        return v.astype(jnp.bfloat16).astype(jnp.float32)
    cross = b16(xb) * b16(x) + b16(yb) * b16(y) + b16(zb) * b16(z)
    d = (sqb + sq) - 2.0 * cross

    iota = lax.broadcasted_iota(jnp.int32, (rows, N), 1)
    kcol = lax.broadcasted_iota(jnp.int32, (rows, KPAD), 1)
    vals = jnp.zeros((rows, KPAD), jnp.float32)
    idxs = jnp.zeros((rows, KPAD), jnp.int32)
    for k in range(K):
        m = jnp.min(d, axis=1, keepdims=True)
        cand = jnp.where(d == m, iota, N)
        j = jnp.min(cand, axis=1, keepdims=True)
        vals = jnp.where(kcol == k, m, vals)
        idxs = jnp.where(kcol == k, j, idxs)
        d = jnp.where(cand == j, BIG, d)
    vals_ref[...] = vals
    idx_ref[...] = idxs


def kernel(pos):
    rows = 256
    grid = N // rows
    posT = pos.T
    vals, idxs = pl.pallas_call(
        functools.partial(_knn_body, rows=rows),
        grid=(grid,),
        in_specs=[
            pl.BlockSpec((rows, 3), lambda i: (i, 0)),
            pl.BlockSpec((3, N), lambda i: (0, 0)),
        ],
        out_specs=[
            pl.BlockSpec((rows, KPAD), lambda i: (i, 0)),
            pl.BlockSpec((rows, KPAD), lambda i: (i, 0)),
        ],
        out_shape=[
            jax.ShapeDtypeStruct((N, KPAD), jnp.float32),
            jax.ShapeDtypeStruct((N, KPAD), jnp.int32),
        ],
    )(pos, posT)
    knn_d2 = vals[:, :K]
    sources = idxs[:, :K].reshape(-1)
    targets = jnp.repeat(jnp.arange(N, dtype=jnp.int32), K)
    edge_index = jnp.stack([sources, targets], axis=0)
    return edge_index, knn_d2


# speedup vs baseline: 13.0159x; 1.1951x over previous
import functools

import jax
import jax.numpy as jnp
from jax import lax
from jax.experimental import pallas as pl

N = 4096
K = 9
KPAD = 16
BIG = 3.0e38


def _knn_body(pos_ref, posT_ref, vals_ref, idx_ref, *, rows):
    xb = pos_ref[:, 0:1]
    yb = pos_ref[:, 1:2]
    zb = pos_ref[:, 2:3]
    x = posT_ref[0:1, :]
    y = posT_ref[1:2, :]
    z = posT_ref[2:3, :]
    sqb = xb * xb + yb * yb + zb * zb
    sq = x * x + y * y + z * z
    def b16(v):
        return v.astype(jnp.bfloat16).astype(jnp.float32)
    cross = b16(xb) * b16(x) + b16(yb) * b16(y) + b16(zb) * b16(z)
    d = (sqb + sq) - 2.0 * cross

    iota_f = lax.broadcasted_iota(jnp.int32, (rows, N), 1).astype(jnp.float32)
    kcol = lax.broadcasted_iota(jnp.int32, (rows, KPAD), 1)
    vals = jnp.zeros((rows, KPAD), jnp.float32)
    idxs = jnp.zeros((rows, KPAD), jnp.int32)
    for k in range(K):
        m = jnp.min(d, axis=1, keepdims=True)
        cand = jnp.where(d == m, iota_f, BIG)
        jf = jnp.min(cand, axis=1, keepdims=True)
        vals = jnp.where(kcol == k, m, vals)
        idxs = jnp.where(kcol == k, jf.astype(jnp.int32), idxs)
        d = jnp.where(cand == jf, BIG, d)
    vals_ref[...] = vals
    idx_ref[...] = idxs


def kernel(pos):
    rows = 256
    grid = N // rows
    posT = pos.T
    vals, idxs = pl.pallas_call(
        functools.partial(_knn_body, rows=rows),
        grid=(grid,),
        in_specs=[
            pl.BlockSpec((rows, 3), lambda i: (i, 0)),
            pl.BlockSpec((3, N), lambda i: (0, 0)),
        ],
        out_specs=[
            pl.BlockSpec((rows, KPAD), lambda i: (i, 0)),
            pl.BlockSpec((rows, KPAD), lambda i: (i, 0)),
        ],
        out_shape=[
            jax.ShapeDtypeStruct((N, KPAD), jnp.float32),
            jax.ShapeDtypeStruct((N, KPAD), jnp.int32),
        ],
    )(pos, posT)
    knn_d2 = vals[:, :K]
    sources = idxs[:, :K].reshape(-1)
    targets = jnp.repeat(jnp.arange(N, dtype=jnp.int32), K)
    edge_index = jnp.stack([sources, targets], axis=0)
    return edge_index, knn_d2


# speedup vs baseline: 13.0873x; 1.0055x over previous
import functools

import jax
import jax.numpy as jnp
from jax import lax
from jax.experimental import pallas as pl

N = 4096
K = 9
KPAD = 16
BIG = 3.0e38


def _knn_body(pos_ref, posT_ref, vals_ref, idx_ref, *, rows):
    xb = pos_ref[:, 0:1]
    yb = pos_ref[:, 1:2]
    zb = pos_ref[:, 2:3]
    x = posT_ref[0:1, :]
    y = posT_ref[1:2, :]
    z = posT_ref[2:3, :]
    sqb = xb * xb + yb * yb + zb * zb
    sq = x * x + y * y + z * z
    def b16(v):
        return v.astype(jnp.bfloat16).astype(jnp.float32)
    cross = b16(xb) * b16(x) + b16(yb) * b16(y) + b16(zb) * b16(z)
    d = (sqb + sq) - 2.0 * cross

    iota_f = lax.broadcasted_iota(jnp.int32, (rows, N), 1).astype(jnp.float32)
    kcol = lax.broadcasted_iota(jnp.int32, (rows, KPAD), 1)
    vals = jnp.zeros((rows, KPAD), jnp.float32)
    idxs = jnp.zeros((rows, KPAD), jnp.int32)
    for k in range(K):
        m = jnp.min(d, axis=1, keepdims=True)
        cand = jnp.where(d == m, iota_f, BIG)
        jf = jnp.min(cand, axis=1, keepdims=True)
        vals = jnp.where(kcol == k, m, vals)
        idxs = jnp.where(kcol == k, jf.astype(jnp.int32), idxs)
        d = jnp.where(cand == jf, BIG, d)
    vals_ref[...] = vals
    idx_ref[...] = idxs


def kernel(pos):
    rows = 512
    grid = N // rows
    posT = pos.T
    vals, idxs = pl.pallas_call(
        functools.partial(_knn_body, rows=rows),
        grid=(grid,),
        in_specs=[
            pl.BlockSpec((rows, 3), lambda i: (i, 0)),
            pl.BlockSpec((3, N), lambda i: (0, 0)),
        ],
        out_specs=[
            pl.BlockSpec((rows, KPAD), lambda i: (i, 0)),
            pl.BlockSpec((rows, KPAD), lambda i: (i, 0)),
        ],
        out_shape=[
            jax.ShapeDtypeStruct((N, KPAD), jnp.float32),
            jax.ShapeDtypeStruct((N, KPAD), jnp.int32),
        ],
    )(pos, posT)
    knn_d2 = vals[:, :K]
    sources = idxs[:, :K].reshape(-1)
    targets = jnp.repeat(jnp.arange(N, dtype=jnp.int32), K)
    edge_index = jnp.stack([sources, targets], axis=0)
    return edge_index, knn_d2


# speedup vs baseline: 14.1219x; 1.0791x over previous
import functools

import jax
import jax.numpy as jnp
from jax import lax
from jax.experimental import pallas as pl

N = 4096
K = 9
KPAD = 16
BIG = 3.0e38


def _knn_body(pos_ref, posT_ref, vals_ref, idx_ref, *, rows):
    xb = pos_ref[:, 0:1]
    yb = pos_ref[:, 1:2]
    zb = pos_ref[:, 2:3]
    x = posT_ref[0:1, :]
    y = posT_ref[1:2, :]
    z = posT_ref[2:3, :]
    sqb = xb * xb + yb * yb + zb * zb
    sq = x * x + y * y + z * z
    cross = lax.dot_general(pos_ref[...].astype(jnp.bfloat16),
                            posT_ref[...].astype(jnp.bfloat16),
                            (((1,), (0,)), ((), ())),
                            preferred_element_type=jnp.float32)
    d = (sqb + sq) - 2.0 * cross

    iota_f = lax.broadcasted_iota(jnp.int32, (rows, N), 1).astype(jnp.float32)
    kcol = lax.broadcasted_iota(jnp.int32, (rows, KPAD), 1)
    vals = jnp.zeros((rows, KPAD), jnp.float32)
    idxs = jnp.zeros((rows, KPAD), jnp.int32)
    for k in range(K):
        m = jnp.min(d, axis=1, keepdims=True)
        cand = jnp.where(d == m, iota_f, BIG)
        jf = jnp.min(cand, axis=1, keepdims=True)
        vals = jnp.where(kcol == k, m, vals)
        idxs = jnp.where(kcol == k, jf.astype(jnp.int32), idxs)
        d = jnp.where(cand == jf, BIG, d)
    vals_ref[...] = vals
    idx_ref[...] = idxs


def kernel(pos):
    rows = 512
    grid = N // rows
    posT = pos.T
    vals, idxs = pl.pallas_call(
        functools.partial(_knn_body, rows=rows),
        grid=(grid,),
        in_specs=[
            pl.BlockSpec((rows, 3), lambda i: (i, 0)),
            pl.BlockSpec((3, N), lambda i: (0, 0)),
        ],
        out_specs=[
            pl.BlockSpec((rows, KPAD), lambda i: (i, 0)),
            pl.BlockSpec((rows, KPAD), lambda i: (i, 0)),
        ],
        out_shape=[
            jax.ShapeDtypeStruct((N, KPAD), jnp.float32),
            jax.ShapeDtypeStruct((N, KPAD), jnp.int32),
        ],
    )(pos, posT)
    knn_d2 = vals[:, :K]
    sources = idxs[:, :K].reshape(-1)
    targets = jnp.repeat(jnp.arange(N, dtype=jnp.int32), K)
    edge_index = jnp.stack([sources, targets], axis=0)
    return edge_index, knn_d2
